# baseline (speedup 1.0000x reference)
import functools

import jax
import jax.numpy as jnp
from jax import lax
from jax.experimental import pallas as pl
from jax.experimental.pallas import tpu as pltpu
from jax.experimental.pallas import tpu_sc as plsc

N_BINS = 15
NW = 32


def _sc_hist_body(n_rows, n_classes, base_blocks, extra, slots_pad,
                  logits_hbm, labels_hbm,
                  cnt_out, cnf_out, acc_out, ml_out,
                  labels_v, buf, east, cnt_h, cnf_h, acc_h, red, mlv):
    w = lax.axis_index("s") * 2 + lax.axis_index("c")
    iota = lax.iota(jnp.int32, 16)
    laneoff = iota * slots_pad
    groups = slots_pad // 16
    nmax = 16 * (base_blocks + 1)
    nmin = 16 * base_blocks

    base = 16 * (w * base_blocks + jnp.minimum(w, extra))
    nblk = base_blocks + (w < extra).astype(jnp.int32)

    zero16 = jnp.zeros((16,), jnp.float32)
    ones16 = jnp.ones((16,), jnp.float32)

    def zero_body(g, carry):
        off = g * 16
        cnt_h[pl.ds(off, 16)] = zero16
        cnf_h[pl.ds(off, 16)] = zero16
        acc_h[pl.ds(off, 16)] = zero16
        return carry

    lax.fori_loop(0, 16 * groups, zero_body, 0)

    @pl.when(w < extra)
    def _labels_big():
        pltpu.sync_copy(labels_hbm.at[pl.ds(base, nmax)], labels_v)

    @pl.when(w >= extra)
    def _labels_small():
        pltpu.sync_copy(labels_hbm.at[pl.ds(base, nmin)],
                        labels_v.at[pl.ds(0, nmin)])

    def bin_of(t):
        ti = t.astype(jnp.int32)
        on_edge = (t == ti.astype(jnp.float32)).astype(jnp.int32)
        return jnp.clip(ti - on_edge, 0, N_BINS - 1)

    def block_body(blk, ml):
        row0 = base + blk * 16
        pltpu.sync_copy(logits_hbm.at[pl.ds(row0 * n_classes, 16 * n_classes)],
                        buf)

        def pass_a(c, s):
            v = plsc.load_gather(buf, [iota * n_classes + c])
            e = jnp.exp(v)
            east[pl.ds(c * 16, 16)] = e
            return s + e

        s = lax.fori_loop(0, n_classes, pass_a, zero16)
        rinv = 1.0 / s

        def pass_b(c, carry):
            e = east[pl.ds(c * 16, 16)]
            p = e * rinv
            bi = bin_of(p * float(N_BINS))
            idx = laneoff + c * N_BINS + bi
            m = p > 0.0
            plsc.addupdate_scatter(cnt_h, [idx], ones16, mask=m)
            plsc.addupdate_scatter(cnf_h, [idx], p, mask=m)
            return carry

        lax.fori_loop(0, n_classes, pass_b, 0)

        lab = labels_v[pl.ds(blk * 16, 16)]
        e_l = plsc.load_gather(east, [lab * 16 + iota])
        p_l = e_l * rinv
        bi = bin_of(p_l * float(N_BINS))
        idx = laneoff + lab * N_BINS + bi
        plsc.addupdate_scatter(acc_h, [idx], ones16, mask=p_l > 0.0)
        return jnp.maximum(ml, lab)

    ml = lax.fori_loop(0, nblk, block_body, jnp.zeros((16,), jnp.int32))
    mlv[...] = ml
    pltpu.sync_copy(mlv, ml_out.at[pl.ds(w * 16, 16)])

    for hist, out in ((cnt_h, cnt_out), (cnf_h, cnf_out), (acc_h, acc_out)):
        def red_body(g, carry, hist=hist):
            a = zero16
            for l in range(16):
                a = a + hist[pl.ds(l * slots_pad + g * 16, 16)]
            red[pl.ds(g * 16, 16)] = a
            return carry

        lax.fori_loop(0, groups, red_body, 0)
        pltpu.sync_copy(red, out.at[pl.ds(w * slots_pad, slots_pad)])


def _sc_hist(logits, labels):
    n_rows, n_classes = logits.shape
    assert n_rows % 16 == 0
    blocks_total = n_rows // 16
    base_blocks = blocks_total // NW
    extra = blocks_total % NW
    slots = N_BINS * n_classes
    slots_pad = ((slots + 15) // 16) * 16
    nmax = 16 * (base_blocks + 1)

    mesh = plsc.VectorSubcoreMesh(core_axis_name="c", subcore_axis_name="s")
    body = functools.partial(_sc_hist_body, n_rows, n_classes,
                             base_blocks, extra, slots_pad)
    f = pl.kernel(
        body,
        mesh=mesh,
        compiler_params=pltpu.CompilerParams(needs_layout_passes=False),
        out_type=[
            jax.ShapeDtypeStruct((NW * slots_pad,), jnp.float32),
            jax.ShapeDtypeStruct((NW * slots_pad,), jnp.float32),
            jax.ShapeDtypeStruct((NW * slots_pad,), jnp.float32),
            jax.ShapeDtypeStruct((NW * 16,), jnp.int32),
        ],
        scratch_types=[
            pltpu.VMEM((nmax,), jnp.int32),
            pltpu.VMEM((16 * n_classes,), jnp.float32),
            pltpu.VMEM((16 * n_classes,), jnp.float32),
            pltpu.VMEM((16 * slots_pad,), jnp.float32),
            pltpu.VMEM((16 * slots_pad,), jnp.float32),
            pltpu.VMEM((16 * slots_pad,), jnp.float32),
            pltpu.VMEM((slots_pad,), jnp.float32),
            pltpu.VMEM((16,), jnp.int32),
        ],
    )
    cnt, cnf, acc, ml = f(logits.reshape(-1), labels)
    return (cnt.reshape(NW, slots_pad), cnf.reshape(NW, slots_pad),
            acc.reshape(NW, slots_pad), ml.reshape(NW, 16))


def _final_body(n_total, n_classes, cnt_ref, cnf_ref, acc_ref, ml_ref,
                out_ref):
    cnt = jnp.sum(cnt_ref[...], axis=0, keepdims=True)
    cnf = jnp.sum(cnf_ref[...], axis=0, keepdims=True)
    acc = jnp.sum(acc_ref[...], axis=0, keepdims=True)
    nc = jnp.max(ml_ref[...]) + 1
    nonempty = cnt > 0.0
    denom = jnp.maximum(cnt, 1.0)
    avg_conf = jnp.where(nonempty, cnf / denom, 0.0)
    avg_acc = jnp.where(nonempty, acc / denom, 0.0)
    prop = cnt / jnp.float32(n_total)
    s_iota = lax.broadcasted_iota(jnp.int32, cnt.shape, 1)
    valid = nonempty & (s_iota < nc * N_BINS)
    contrib = jnp.where(valid, jnp.abs(avg_conf - avg_acc) * prop, 0.0)
    total = jnp.sum(contrib, axis=(0, 1), keepdims=True)
    out_ref[...] = total / nc.astype(jnp.float32)


def kernel(logits, labels):
    n_rows, n_classes = logits.shape
    cnt, cnf, acc, ml = _sc_hist(logits, labels)
    out = pl.pallas_call(
        functools.partial(_final_body, n_rows, n_classes),
        out_shape=jax.ShapeDtypeStruct((1, 1), jnp.float32),
    )(cnt, cnf, acc, ml)
    return out[0, 0]


# speedup vs baseline: 1.7077x; 1.7077x over previous
import functools

import jax
import jax.numpy as jnp
from jax import lax
from jax.experimental import pallas as pl
from jax.experimental.pallas import tpu as pltpu
from jax.experimental.pallas import tpu_sc as plsc

N_BINS = 15
NW = 32
NBUF = 4
UNROLL = 4


def _sc_hist_body(n_rows, n_classes, base_blocks, extra, rep_stride,
                  logits_hbm, labels_hbm,
                  cnt_out, cnf_out, acc_out, ml_out,
                  labels_v, buf, east, cnt_h, cnf_h, acc_h, red, mlv, *sems):
    w = lax.axis_index("s") * 2 + lax.axis_index("c")
    iota = lax.iota(jnp.int32, 16)
    laneoff = iota * rep_stride
    slots_pad = rep_stride - 1
    groups = slots_pad // 16
    nmax = 16 * (base_blocks + 1)
    nmin = 16 * base_blocks
    nblk_max = base_blocks + (1 if extra else 0)

    base = 16 * (w * base_blocks + jnp.minimum(w, extra))
    nblk = base_blocks + (w < extra).astype(jnp.int32)

    zero16 = jnp.zeros((16,), jnp.float32)
    ones16 = jnp.ones((16,), jnp.float32)

    def zero_body(g, carry):
        off = g * 16
        cnt_h[pl.ds(off, 16)] = zero16
        cnf_h[pl.ds(off, 16)] = zero16
        acc_h[pl.ds(off, 16)] = zero16
        return carry

    lax.fori_loop(0, 16 * rep_stride // 16, zero_body, 0)

    @pl.when(w < extra)
    def _labels_big():
        pltpu.sync_copy(labels_hbm.at[pl.ds(base, nmax)], labels_v)

    @pl.when(w >= extra)
    def _labels_small():
        pltpu.sync_copy(labels_hbm.at[pl.ds(base, nmin)],
                        labels_v.at[pl.ds(0, nmin)])

    def blk_row0(blk):
        return jnp.minimum(base + blk * 16, n_rows - 16)

    def start_copy(blk, k):
        pltpu.async_copy(logits_hbm.at[pl.ds(blk_row0(blk), 16)],
                         buf.at[k], sems[k])

    def wait_copy(blk, k):
        pltpu.make_async_copy(logits_hbm.at[pl.ds(blk_row0(blk), 16)],
                              buf.at[k], sems[k]).wait()

    for k in range(NBUF):
        start_copy(k, k)

    def bin_of(t):
        ti = t.astype(jnp.int32)
        on_edge = (t == ti.astype(jnp.float32)).astype(jnp.int32)
        return jnp.clip(ti - on_edge, 0, N_BINS - 1)

    def block_compute(blk, k, ml):
        real = blk < nblk
        bufk = buf.at[k]

        def pass_a(ci, s):
            for u in range(UNROLL):
                c = ci * UNROLL + u
                v = plsc.load_gather(bufk, [iota, jnp.zeros((16,), jnp.int32) + c])
                e = jnp.exp(v)
                east[pl.ds(c * 16, 16)] = e
                s = s + e
            return s

        s = lax.fori_loop(0, n_classes // UNROLL, pass_a, zero16)
        rinv = 1.0 / s

        def pass_b(ci, carry):
            for u in range(UNROLL):
                c = ci * UNROLL + u
                e = east[pl.ds(c * 16, 16)]
                p = e * rinv
                bi = bin_of(p * float(N_BINS))
                idx = laneoff + (c * N_BINS + bi)
                m = (p > 0.0) & real
                plsc.addupdate_scatter(cnt_h, [idx], ones16, mask=m)
                plsc.addupdate_scatter(cnf_h, [idx], p, mask=m)
            return carry

        lax.fori_loop(0, n_classes // UNROLL, pass_b, 0)

        lab = jnp.clip(labels_v[pl.ds(blk * 16, 16)], 0, n_classes - 1)
        e_l = plsc.load_gather(east, [lab * 16 + iota])
        p_l = e_l * rinv
        bi = bin_of(p_l * float(N_BINS))
        idx = laneoff + (lab * N_BINS + bi)
        plsc.addupdate_scatter(acc_h, [idx], ones16, mask=(p_l > 0.0) & real)
        return jnp.where(real, jnp.maximum(ml, lab), ml)

    def group_body(g, ml):
        for k in range(NBUF):
            blk = g * NBUF + k
            wait_copy(blk, k)
            ml = block_compute(blk, k, ml)
            nxt = blk + NBUF

            @pl.when(nxt < nblk_max)
            def _():
                start_copy(nxt, k)
        return ml

    assert nblk_max % NBUF == 0
    ml = lax.fori_loop(0, nblk_max // NBUF, group_body,
                       jnp.zeros((16,), jnp.int32))
    mlv[...] = ml
    pltpu.sync_copy(mlv, ml_out.at[pl.ds(w * 16, 16)])

    for hist, out in ((cnt_h, cnt_out), (cnf_h, cnf_out), (acc_h, acc_out)):
        def red_body(g, carry, hist=hist):
            a = zero16
            for l in range(16):
                a = a + plsc.load_gather(hist, [l * rep_stride + g * 16 + iota])
            red[pl.ds(g * 16, 16)] = a
            return carry

        lax.fori_loop(0, groups, red_body, 0)
        pltpu.sync_copy(red, out.at[pl.ds(w * slots_pad, slots_pad)])


def _sc_hist(logits, labels):
    n_rows, n_classes = logits.shape
    assert n_rows % 16 == 0 and n_classes % UNROLL == 0
    blocks_total = n_rows // 16
    base_blocks = blocks_total // NW
    extra = blocks_total % NW
    slots = N_BINS * n_classes
    slots_pad = ((slots + 15) // 16) * 16
    rep_stride = slots_pad + 1
    nmax = 16 * (base_blocks + 1)

    mesh = plsc.VectorSubcoreMesh(core_axis_name="c", subcore_axis_name="s")
    body = functools.partial(_sc_hist_body, n_rows, n_classes,
                             base_blocks, extra, rep_stride)
    f = pl.kernel(
        body,
        mesh=mesh,
        compiler_params=pltpu.CompilerParams(needs_layout_passes=False),
        out_type=[
            jax.ShapeDtypeStruct((NW * slots_pad,), jnp.float32),
            jax.ShapeDtypeStruct((NW * slots_pad,), jnp.float32),
            jax.ShapeDtypeStruct((NW * slots_pad,), jnp.float32),
            jax.ShapeDtypeStruct((NW * 16,), jnp.int32),
        ],
        scratch_types=[
            pltpu.VMEM((nmax,), jnp.int32),
            pltpu.VMEM((NBUF, 16, n_classes), jnp.float32),
            pltpu.VMEM((16 * n_classes,), jnp.float32),
            pltpu.VMEM((16 * rep_stride,), jnp.float32),
            pltpu.VMEM((16 * rep_stride,), jnp.float32),
            pltpu.VMEM((16 * rep_stride,), jnp.float32),
            pltpu.VMEM((slots_pad,), jnp.float32),
            pltpu.VMEM((16,), jnp.int32),
        ] + [pltpu.SemaphoreType.DMA] * NBUF,
    )
    cnt, cnf, acc, ml = f(logits, labels)
    return (cnt.reshape(NW, slots_pad), cnf.reshape(NW, slots_pad),
            acc.reshape(NW, slots_pad), ml.reshape(NW, 16))


def _final_body(n_total, n_classes, cnt_ref, cnf_ref, acc_ref, ml_ref,
                out_ref):
    cnt = jnp.sum(cnt_ref[...], axis=0, keepdims=True)
    cnf = jnp.sum(cnf_ref[...], axis=0, keepdims=True)
    acc = jnp.sum(acc_ref[...], axis=0, keepdims=True)
    nc = jnp.max(ml_ref[...]) + 1
    nonempty = cnt > 0.0
    denom = jnp.maximum(cnt, 1.0)
    avg_conf = jnp.where(nonempty, cnf / denom, 0.0)
    avg_acc = jnp.where(nonempty, acc / denom, 0.0)
    prop = cnt / jnp.float32(n_total)
    s_iota = lax.broadcasted_iota(jnp.int32, cnt.shape, 1)
    valid = nonempty & (s_iota < nc * N_BINS)
    contrib = jnp.where(valid, jnp.abs(avg_conf - avg_acc) * prop, 0.0)
    total = jnp.sum(contrib, axis=(0, 1), keepdims=True)
    out_ref[...] = total / nc.astype(jnp.float32)


def kernel(logits, labels):
    n_rows, n_classes = logits.shape
    cnt, cnf, acc, ml = _sc_hist(logits, labels)
    out = pl.pallas_call(
        functools.partial(_final_body, n_rows, n_classes),
        out_shape=jax.ShapeDtypeStruct((1, 1), jnp.float32),
    )(cnt, cnf, acc, ml)
    return out[0, 0]


# speedup vs baseline: 4.5173x; 2.6452x over previous
import functools

import jax
import jax.numpy as jnp
from jax import lax
from jax.experimental import pallas as pl
from jax.experimental.pallas import tpu as pltpu
from jax.experimental.pallas import tpu_sc as plsc

N_BINS = 15
NW = 32
NBUF = 4
UNROLL = 10


def _sc_hist_body(n_rows, n_classes, base_blocks, extra, rep_stride,
                  logits_hbm, labels_hbm,
                  cnt_out, cnf_out, acc_out, ml_out,
                  labels_v, buf, east, cnt_h, cnf_h, acc_h, red, mlv, *sems):
    w = lax.axis_index("s") * 2 + lax.axis_index("c")
    iota = lax.iota(jnp.int32, 16)
    laneoff = iota * rep_stride
    slots_pad = rep_stride - 1
    groups = slots_pad // 16
    nmax = 16 * (base_blocks + 1)
    nmin = 16 * base_blocks
    nblk_max = base_blocks + (1 if extra else 0)

    base = 16 * (w * base_blocks + jnp.minimum(w, extra))
    nblk = base_blocks + (w < extra).astype(jnp.int32)

    zero16 = jnp.zeros((16,), jnp.float32)
    ones16 = jnp.ones((16,), jnp.float32)

    def zero_body(g, carry):
        off = g * 16
        cnt_h[pl.ds(off, 16)] = zero16
        cnf_h[pl.ds(off, 16)] = zero16
        acc_h[pl.ds(off, 16)] = zero16
        return carry

    lax.fori_loop(0, 16 * rep_stride // 16, zero_body, 0)

    @pl.when(w < extra)
    def _labels_big():
        pltpu.sync_copy(labels_hbm.at[pl.ds(base, nmax)], labels_v)

    @pl.when(w >= extra)
    def _labels_small():
        pltpu.sync_copy(labels_hbm.at[pl.ds(base, nmin)],
                        labels_v.at[pl.ds(0, nmin)])

    def blk_row0(blk):
        return jnp.minimum(base + blk * 16, n_rows - 16)

    def start_copy(blk, k):
        pltpu.async_copy(logits_hbm.at[pl.ds(blk_row0(blk), 16)],
                         buf.at[k], sems[k])

    def wait_copy(blk, k):
        pltpu.make_async_copy(logits_hbm.at[pl.ds(blk_row0(blk), 16)],
                              buf.at[k], sems[k]).wait()

    for k in range(NBUF):
        start_copy(k, k)

    def bin_of(t):
        ti = t.astype(jnp.int32)
        on_edge = (t == ti.astype(jnp.float32)).astype(jnp.int32)
        return jnp.clip(ti - on_edge, 0, N_BINS - 1)

    def block_compute(blk, k, ml):
        real = blk < nblk
        bufk = buf.at[k]

        @plsc.parallel_loop(0, n_classes, unroll=UNROLL, carry=zero16)
        def s(c, s_in):
            v = plsc.load_gather(bufk, [iota, jnp.zeros((16,), jnp.int32) + c])
            e = jnp.exp(v)
            east[pl.ds(c * 16, 16)] = e
            return s_in + e

        rinv = 1.0 / s

        @plsc.parallel_loop(0, n_classes, unroll=UNROLL)
        def _scatter(c):
            e = east[pl.ds(c * 16, 16)]
            p = e * rinv
            bi = bin_of(p * float(N_BINS))
            idx = laneoff + (c * N_BINS + bi)
            m = (p > 0.0) & real
            plsc.addupdate_scatter(cnt_h, [idx], ones16, mask=m)
            plsc.addupdate_scatter(cnf_h, [idx], p, mask=m)

        lab = jnp.clip(labels_v[pl.ds(blk * 16, 16)], 0, n_classes - 1)
        e_l = plsc.load_gather(east, [lab * 16 + iota])
        p_l = e_l * rinv
        bi = bin_of(p_l * float(N_BINS))
        idx = laneoff + (lab * N_BINS + bi)
        plsc.addupdate_scatter(acc_h, [idx], ones16, mask=(p_l > 0.0) & real)
        return jnp.where(real, jnp.maximum(ml, lab), ml)

    def group_body(g, ml):
        for k in range(NBUF):
            blk = g * NBUF + k
            wait_copy(blk, k)
            ml = block_compute(blk, k, ml)
            nxt = blk + NBUF

            @pl.when(nxt < nblk_max)
            def _():
                start_copy(nxt, k)
        return ml

    assert nblk_max % NBUF == 0
    ml = lax.fori_loop(0, nblk_max // NBUF, group_body,
                       jnp.zeros((16,), jnp.int32))
    mlv[...] = ml
    pltpu.sync_copy(mlv, ml_out.at[pl.ds(w * 16, 16)])

    for hist, out in ((cnt_h, cnt_out), (cnf_h, cnf_out), (acc_h, acc_out)):
        def red_body(g, carry, hist=hist):
            a = zero16
            for l in range(16):
                a = a + plsc.load_gather(hist, [l * rep_stride + g * 16 + iota])
            red[pl.ds(g * 16, 16)] = a
            return carry

        lax.fori_loop(0, groups, red_body, 0)
        pltpu.sync_copy(red, out.at[pl.ds(w * slots_pad, slots_pad)])


def _sc_hist(logits, labels):
    n_rows, n_classes = logits.shape
    assert n_rows % 16 == 0 and n_classes % UNROLL == 0
    blocks_total = n_rows // 16
    base_blocks = blocks_total // NW
    extra = blocks_total % NW
    slots = N_BINS * n_classes
    slots_pad = ((slots + 15) // 16) * 16
    rep_stride = slots_pad + 1
    nmax = 16 * (base_blocks + 1)

    mesh = plsc.VectorSubcoreMesh(core_axis_name="c", subcore_axis_name="s")
    body = functools.partial(_sc_hist_body, n_rows, n_classes,
                             base_blocks, extra, rep_stride)
    f = pl.kernel(
        body,
        mesh=mesh,
        compiler_params=pltpu.CompilerParams(needs_layout_passes=False),
        out_type=[
            jax.ShapeDtypeStruct((NW * slots_pad,), jnp.float32),
            jax.ShapeDtypeStruct((NW * slots_pad,), jnp.float32),
            jax.ShapeDtypeStruct((NW * slots_pad,), jnp.float32),
            jax.ShapeDtypeStruct((NW * 16,), jnp.int32),
        ],
        scratch_types=[
            pltpu.VMEM((nmax,), jnp.int32),
            pltpu.VMEM((NBUF, 16, n_classes), jnp.float32),
            pltpu.VMEM((16 * n_classes,), jnp.float32),
            pltpu.VMEM((16 * rep_stride,), jnp.float32),
            pltpu.VMEM((16 * rep_stride,), jnp.float32),
            pltpu.VMEM((16 * rep_stride,), jnp.float32),
            pltpu.VMEM((slots_pad,), jnp.float32),
            pltpu.VMEM((16,), jnp.int32),
        ] + [pltpu.SemaphoreType.DMA] * NBUF,
    )
    cnt, cnf, acc, ml = f(logits, labels)
    return (cnt.reshape(NW, slots_pad), cnf.reshape(NW, slots_pad),
            acc.reshape(NW, slots_pad), ml.reshape(NW, 16))


def _final_body(n_total, n_classes, cnt_ref, cnf_ref, acc_ref, ml_ref,
                out_ref):
    cnt = jnp.sum(cnt_ref[...], axis=0, keepdims=True)
    cnf = jnp.sum(cnf_ref[...], axis=0, keepdims=True)
    acc = jnp.sum(acc_ref[...], axis=0, keepdims=True)
    nc = jnp.max(ml_ref[...]) + 1
    nonempty = cnt > 0.0
    denom = jnp.maximum(cnt, 1.0)
    avg_conf = jnp.where(nonempty, cnf / denom, 0.0)
    avg_acc = jnp.where(nonempty, acc / denom, 0.0)
    prop = cnt / jnp.float32(n_total)
    s_iota = lax.broadcasted_iota(jnp.int32, cnt.shape, 1)
    valid = nonempty & (s_iota < nc * N_BINS)
    contrib = jnp.where(valid, jnp.abs(avg_conf - avg_acc) * prop, 0.0)
    total = jnp.sum(contrib, axis=(0, 1), keepdims=True)
    out_ref[...] = total / nc.astype(jnp.float32)


def kernel(logits, labels):
    n_rows, n_classes = logits.shape
    cnt, cnf, acc, ml = _sc_hist(logits, labels)
    out = pl.pallas_call(
        functools.partial(_final_body, n_rows, n_classes),
        out_shape=jax.ShapeDtypeStruct((1, 1), jnp.float32),
    )(cnt, cnf, acc, ml)
    return out[0, 0]
